# Initial kernel scaffold; baseline (speedup 1.0000x reference)
import jax
import jax.numpy as jnp
from jax.experimental import pallas as pl


def kernel(x, enc1_w, enc1_b, enc2_w, enc2_b, dec1_w, dec1_b, dec2_w, dec2_b):
    raise NotImplementedError("write your pallas kernel here")


# speedup vs baseline: 1.6382x; 1.6382x over previous
import jax
import jax.numpy as jnp
from jax import lax
from jax.experimental import pallas as pl
from jax.experimental.pallas import tpu as pltpu

KP = 5
PW = 8
H = W = 64
C1, C2 = 64, 128
CL = 2 * C1
HP, WP = H // PW, W // PW
K2 = KP * KP * C1
K2P = 13 * 128
K3 = KP * KP * C2

HIGHEST = jax.lax.Precision.HIGHEST
f32 = jnp.float32


def _pair_kernel(xt_ref, w1_ref, b1_ref, w2_ref, b2_ref, w3_ref, b3_ref,
                 w4_ref, b4_ref, out_ref, lat_ref,
                 xpb, p1pad, idx1, u2pad, imcol2, imcol3, up):
    p1pad[...] = jnp.zeros((WP + 4, WP + 4, CL), f32)
    u2pad[...] = jnp.zeros((2, HP + 4, WP + 4, C2), f32)
    imcol2[:, K2:] = jnp.zeros((2 * HP * WP, K2P - K2), f32)
    up[0:2, :, :] = jnp.zeros((2, W + 4, CL), f32)
    up[H + 2:H + 4, :, :] = jnp.zeros((2, W + 4, CL), f32)
    up[2:2 + H, 0:2, :] = jnp.zeros((H, 2, CL), f32)
    up[2:2 + H, W + 2:W + 4, :] = jnp.zeros((H, 2, CL), f32)

    for v in range(H + 4):
        c0 = jnp.broadcast_to(xt_ref[0][:, v:v + 1], (W + 4, C1))
        c1 = jnp.broadcast_to(xt_ref[1][:, v:v + 1], (W + 4, C1))
        xpb[v, :, :] = jnp.concatenate([c0, c1], axis=-1)

    wcol = lax.broadcasted_iota(jnp.int32, (W, CL), 0)

    for I in range(HP):
        def row_body(r, carry):
            best_v, best_f = carry
            h = I * PW + r
            acc = jnp.zeros((W, CL), f32)
            for dy in range(KP):
                for dx in range(KP):
                    acc = acc + xpb[h + dy, dx:dx + W, :] * w1_ref[dy * KP + dx, :]
            a1 = jnp.maximum(acc + b1_ref[0, :], 0.0)
            take = a1 > best_v
            best_f = jnp.where(take, r * W + wcol, best_f)
            best_v = jnp.where(take, a1, best_v)
            return best_v, best_f

        best_v, best_f = lax.fori_loop(
            0, PW, row_body,
            (jnp.full((W, CL), -1.0, f32), jnp.zeros((W, CL), jnp.int32)))

        v3 = best_v.reshape(WP, PW, CL)
        f3 = best_f.reshape(WP, PW, CL)
        pooled = jnp.max(v3, axis=1)
        sel = jnp.min(jnp.where(v3 == pooled[:, None, :], f3, PW * W), axis=1)
        p1pad[2 + I, 2:2 + WP, :] = pooled
        idx1[I] = sel

    for t in range(KP * KP):
        dy, dx = divmod(t, KP)
        slab = p1pad[dy:dy + HP, dx:dx + WP, :]
        imcol2[0:HP * WP, t * C1:(t + 1) * C1] = (
            slab[:, :, 0:C1].reshape(HP * WP, C1))
        imcol2[HP * WP:2 * HP * WP, t * C1:(t + 1) * C1] = (
            slab[:, :, C1:CL].reshape(HP * WP, C1))
    a2 = jnp.dot(imcol2[...], w2_ref[...],
                 preferred_element_type=f32, precision=HIGHEST) + b2_ref[0, :]
    a2 = jnp.maximum(a2, 0.0)

    piota = lax.broadcasted_iota(jnp.int32, (HP * WP, C2), 0)
    for i in range(2):
        ai = a2[i * HP * WP:(i + 1) * HP * WP, :]
        lat = jnp.max(ai, axis=0)
        lat_ref[i, :] = lat
        idx2 = jnp.min(jnp.where(ai == lat[None, :], piota, HP * WP), axis=0)
        u2 = jnp.where(piota == idx2[None, :], lat[None, :], 0.0)
        u2pad[i, 2:2 + HP, 2:2 + WP, :] = u2.reshape(HP, WP, C2)

    for t in range(KP * KP):
        dy, dx = divmod(t, KP)
        for i in range(2):
            imcol3[i * HP * WP:(i + 1) * HP * WP, t * C2:(t + 1) * C2] = (
                u2pad[i, dy:dy + HP, dx:dx + WP, :].reshape(HP * WP, C2))
    d1 = jnp.dot(imcol3[...], w3_ref[...],
                 preferred_element_type=f32, precision=HIGHEST) + b3_ref[0, :]
    d1 = jnp.maximum(d1, 0.0)
    lane = lax.broadcasted_iota(jnp.int32, (HP * WP, CL), 1)
    d1p = jnp.where(lane < C1, d1[0:HP * WP, :], d1[HP * WP:2 * HP * WP, :])

    riota = lax.broadcasted_iota(jnp.int32, (PW, W, CL), 0)
    wiota = lax.broadcasted_iota(jnp.int32, (PW, W, CL), 1)
    flatpos = riota * W + wiota
    for I in range(HP):
        vals = d1p[I * WP:(I + 1) * WP, :]
        idxs = idx1[I]
        vflat = jnp.broadcast_to(vals[:, None, :], (WP, PW, CL)).reshape(W, CL)
        iflat = jnp.broadcast_to(idxs[:, None, :], (WP, PW, CL)).reshape(W, CL)
        slab = jnp.where(flatpos == iflat[None, :, :], vflat[None, :, :], 0.0)
        up[2 + I * PW:2 + (I + 1) * PW, 2:2 + W, :] = slab

    b4 = b4_ref[0]

    def dec2_body(hp, carry):
        h0 = hp * 2
        rows0, rows1 = [], []
        for rr in range(2):
            acc = jnp.zeros((W, CL), f32)
            for dy in range(KP):
                for dx in range(KP):
                    acc = acc + up[h0 + rr + dy, dx:dx + W, :] * w4_ref[dy * KP + dx, :]
            rows0.append(jnp.sum(acc[:, 0:C1], axis=-1) + b4)
            rows1.append(jnp.sum(acc[:, C1:CL], axis=-1) + b4)
        out_ref[0, hp, :] = jnp.maximum(jnp.concatenate(rows0, axis=0), 0.0)
        out_ref[1, hp, :] = jnp.maximum(jnp.concatenate(rows1, axis=0), 0.0)
        return carry

    lax.fori_loop(0, H // 2, dec2_body, 0)


def _pack_conv_w(w_oihw):
    co, ci = w_oihw.shape[0], w_oihw.shape[1]
    return jnp.transpose(w_oihw, (2, 3, 1, 0)).reshape(KP * KP * ci, co)


def _pack_convT_w(w_iohw):
    ci, co = w_iohw.shape[0], w_iohw.shape[1]
    return jnp.transpose(w_iohw[:, :, ::-1, ::-1], (2, 3, 0, 1)).reshape(KP * KP * ci, co)


def kernel(x, enc1_w, enc1_b, enc2_w, enc2_b, dec1_w, dec1_b, dec2_w, dec2_b):
    N = x.shape[0]
    P = N // 2

    w1 = _pack_conv_w(enc1_w)
    w1p = jnp.concatenate([w1, w1], axis=1)
    b1p = jnp.tile(enc1_b.reshape(1, C1), (1, 2))
    w2 = jnp.pad(_pack_conv_w(enc2_w), ((0, K2P - K2), (0, 0)))
    b2 = enc2_b.reshape(1, C2)
    w3 = _pack_convT_w(dec1_w)
    w3p = jnp.concatenate([w3, w3], axis=1)
    b3p = jnp.tile(dec1_b.reshape(1, C1), (1, 2))
    w4 = _pack_convT_w(dec2_w).reshape(KP * KP, C1)
    w4p = jnp.concatenate([w4, w4], axis=1)
    b4 = dec2_b.reshape(1)

    xpad = jnp.pad(x[:, 0], ((0, 0), (2, 2), (2, 2)))
    xt = jnp.transpose(xpad, (0, 2, 1)).reshape(P, 2, H + 4, W + 4)

    flops = N * 2 * (H * W * KP * KP * C1 + HP * WP * K2 * C2
                     + HP * WP * K3 * C1 + H * W * KP * KP * C1)
    bytes_accessed = 4 * (xt.size + w1p.size + w2.size + w3p.size + w4p.size
                          + N * (H * W + C2))

    recon_pk, latent = pl.pallas_call(
        _pair_kernel,
        out_shape=(jax.ShapeDtypeStruct((P, 2, H // 2, 2 * W), f32),
                   jax.ShapeDtypeStruct((P, 2, C2), f32)),
        grid=(P,),
        in_specs=[
            pl.BlockSpec((None, 2, H + 4, W + 4), lambda p: (p, 0, 0, 0)),
            pl.BlockSpec((KP * KP, CL), lambda p: (0, 0)),
            pl.BlockSpec((1, CL), lambda p: (0, 0)),
            pl.BlockSpec((K2P, C2), lambda p: (0, 0)),
            pl.BlockSpec((1, C2), lambda p: (0, 0)),
            pl.BlockSpec((K3, CL), lambda p: (0, 0)),
            pl.BlockSpec((1, CL), lambda p: (0, 0)),
            pl.BlockSpec((KP * KP, CL), lambda p: (0, 0)),
            pl.BlockSpec(memory_space=pltpu.MemorySpace.SMEM),
        ],
        out_specs=(pl.BlockSpec((None, 2, H // 2, 2 * W), lambda p: (p, 0, 0, 0)),
                   pl.BlockSpec((None, 2, C2), lambda p: (p, 0, 0))),
        scratch_shapes=[
            pltpu.VMEM((H + 4, W + 4, CL), f32),
            pltpu.VMEM((WP + 4, WP + 4, CL), f32),
            pltpu.VMEM((HP, WP, CL), jnp.int32),
            pltpu.VMEM((2, HP + 4, WP + 4, C2), f32),
            pltpu.VMEM((2 * HP * WP, K2P), f32),
            pltpu.VMEM((2 * HP * WP, K3), f32),
            pltpu.VMEM((H + 4, W + 4, CL), f32),
        ],
        compiler_params=pltpu.CompilerParams(
            dimension_semantics=("parallel",),
            vmem_limit_bytes=32 * 1024 * 1024),
        cost_estimate=pl.CostEstimate(flops=flops, transcendentals=0,
                                      bytes_accessed=bytes_accessed),
    )(xt, w1p, b1p, w2, b2, w3p, b3p, w4p, b4)

    recon = recon_pk.reshape(N, H, W)[:, None, :, :]
    return recon, latent.reshape(-1)


# speedup vs baseline: 2.3901x; 1.4590x over previous
import jax
import jax.numpy as jnp
from jax import lax
from jax.experimental import pallas as pl
from jax.experimental.pallas import tpu as pltpu

KP = 5
PW = 8
H = W = 64
C1, C2 = 64, 128
CL = 2 * C1
HP, WP = H // PW, W // PW
K2 = KP * KP * C1
K2P = 13 * 128
K3 = KP * KP * C2

HIGHEST = jax.lax.Precision.HIGHEST
f32 = jnp.float32


def _pair_kernel(xt_ref, w1_ref, b1_ref, w2_ref, b2_ref, w3_ref, b3_ref,
                 w4_ref, m2_ref, b4_ref, out_ref, lat_ref,
                 xpb, p1pad, idx1, u2pad, imcol2, imcol3, up, scol):
    p1pad[...] = jnp.zeros((WP + 4, WP + 4, CL), f32)
    u2pad[...] = jnp.zeros((2, HP + 4, WP + 4, C2), f32)
    imcol2[:, K2:] = jnp.zeros((2 * HP * WP, K2P - K2), f32)
    up[0:2, :, :] = jnp.zeros((2, W + 4, CL), f32)
    up[H + 2:H + 4, :, :] = jnp.zeros((2, W + 4, CL), f32)
    up[2:2 + H, 0:2, :] = jnp.zeros((H, 2, CL), f32)
    up[2:2 + H, W + 2:W + 4, :] = jnp.zeros((H, 2, CL), f32)

    for v in range(H + 4):
        c0 = jnp.broadcast_to(xt_ref[0][:, v:v + 1], (W + 4, C1))
        c1 = jnp.broadcast_to(xt_ref[1][:, v:v + 1], (W + 4, C1))
        xpb[v, :, :] = jnp.concatenate([c0, c1], axis=-1)

    wcol = lax.broadcasted_iota(jnp.int32, (W, CL), 0)

    for I in range(HP):
        best_v = jnp.full((W, CL), -1.0, f32)
        best_f = jnp.zeros((W, CL), jnp.int32)
        for r in range(PW):
            h = I * PW + r
            acc = jnp.zeros((W, CL), f32)
            for dy in range(KP):
                for dx in range(KP):
                    acc = acc + xpb[h + dy, dx:dx + W, :] * w1_ref[dy * KP + dx, :]
            a1 = jnp.maximum(acc + b1_ref[0, :], 0.0)
            take = a1 > best_v
            best_f = jnp.where(take, r * W + wcol, best_f)
            best_v = jnp.where(take, a1, best_v)

        v3 = best_v.reshape(WP, PW, CL)
        f3 = best_f.reshape(WP, PW, CL)
        pooled = jnp.max(v3, axis=1)
        sel = jnp.min(jnp.where(v3 == pooled[:, None, :], f3, PW * W), axis=1)
        p1pad[2 + I, 2:2 + WP, :] = pooled
        idx1[I] = sel

    for t in range(KP * KP):
        dy, dx = divmod(t, KP)
        slab = p1pad[dy:dy + HP, dx:dx + WP, :]
        imcol2[0:HP * WP, t * C1:(t + 1) * C1] = (
            slab[:, :, 0:C1].reshape(HP * WP, C1))
        imcol2[HP * WP:2 * HP * WP, t * C1:(t + 1) * C1] = (
            slab[:, :, C1:CL].reshape(HP * WP, C1))
    a2 = jnp.dot(imcol2[...], w2_ref[...],
                 preferred_element_type=f32, precision=HIGHEST) + b2_ref[0, :]
    a2 = jnp.maximum(a2, 0.0)

    piota = lax.broadcasted_iota(jnp.int32, (HP * WP, C2), 0)
    for i in range(2):
        ai = a2[i * HP * WP:(i + 1) * HP * WP, :]
        lat = jnp.max(ai, axis=0)
        lat_ref[i, :] = lat
        idx2 = jnp.min(jnp.where(ai == lat[None, :], piota, HP * WP), axis=0)
        u2 = jnp.where(piota == idx2[None, :], lat[None, :], 0.0)
        u2pad[i, 2:2 + HP, 2:2 + WP, :] = u2.reshape(HP, WP, C2)

    for t in range(KP * KP):
        dy, dx = divmod(t, KP)
        for i in range(2):
            imcol3[i * HP * WP:(i + 1) * HP * WP, t * C2:(t + 1) * C2] = (
                u2pad[i, dy:dy + HP, dx:dx + WP, :].reshape(HP * WP, C2))
    d1 = jnp.dot(imcol3[...], w3_ref[...],
                 preferred_element_type=f32, precision=HIGHEST) + b3_ref[0, :]
    d1 = jnp.maximum(d1, 0.0)
    lane = lax.broadcasted_iota(jnp.int32, (HP * WP, CL), 1)
    d1p = jnp.where(lane < C1, d1[0:HP * WP, :], d1[HP * WP:2 * HP * WP, :])

    riota = lax.broadcasted_iota(jnp.int32, (PW, W, CL), 0)
    wiota = lax.broadcasted_iota(jnp.int32, (PW, W, CL), 1)
    flatpos = riota * W + wiota
    for I in range(HP):
        vals = d1p[I * WP:(I + 1) * WP, :]
        idxs = idx1[I]
        vflat = jnp.broadcast_to(vals[:, None, :], (WP, PW, CL)).reshape(W, CL)
        iflat = jnp.broadcast_to(idxs[:, None, :], (WP, PW, CL)).reshape(W, CL)
        slab = jnp.where(flatpos == iflat[None, :, :], vflat[None, :, :], 0.0)
        up[2 + I * PW:2 + (I + 1) * PW, 2:2 + W, :] = slab

    for h in range(H):
        acc = jnp.zeros((W, CL), f32)
        for dy in range(KP):
            for dx in range(KP):
                acc = acc + up[h + dy, dx:dx + W, :] * w4_ref[dy * KP + dx, :]
        s = jnp.dot(acc, m2_ref[...],
                    preferred_element_type=f32, precision=HIGHEST)
        scol[:, h:h + 1] = s[:, 0:1]
        scol[:, H + h:H + h + 1] = s[:, 1:2]

    b4 = b4_ref[0]
    sc = scol[...]
    t0 = jnp.maximum(jnp.swapaxes(sc[:, 0:H], 0, 1) + b4, 0.0)
    t1 = jnp.maximum(jnp.swapaxes(sc[:, H:2 * H], 0, 1) + b4, 0.0)
    out_ref[0, :, :] = t0
    out_ref[1, :, :] = t1


def _pack_conv_w(w_oihw):
    co, ci = w_oihw.shape[0], w_oihw.shape[1]
    return jnp.transpose(w_oihw, (2, 3, 1, 0)).reshape(KP * KP * ci, co)


def _pack_convT_w(w_iohw):
    ci, co = w_iohw.shape[0], w_iohw.shape[1]
    return jnp.transpose(w_iohw[:, :, ::-1, ::-1], (2, 3, 0, 1)).reshape(KP * KP * ci, co)


def kernel(x, enc1_w, enc1_b, enc2_w, enc2_b, dec1_w, dec1_b, dec2_w, dec2_b):
    N = x.shape[0]
    P = N // 2

    w1 = _pack_conv_w(enc1_w)
    w1p = jnp.concatenate([w1, w1], axis=1)
    b1p = jnp.tile(enc1_b.reshape(1, C1), (1, 2))
    w2 = jnp.pad(_pack_conv_w(enc2_w), ((0, K2P - K2), (0, 0)))
    b2 = enc2_b.reshape(1, C2)
    w3 = _pack_convT_w(dec1_w)
    w3p = jnp.concatenate([w3, w3], axis=1)
    b3p = jnp.tile(dec1_b.reshape(1, C1), (1, 2))
    w4 = _pack_convT_w(dec2_w).reshape(KP * KP, C1)
    w4p = jnp.concatenate([w4, w4], axis=1)
    b4 = dec2_b.reshape(1)
    m2 = jnp.concatenate([
        jnp.concatenate([jnp.ones((C1, 1), f32), jnp.zeros((C1, 1), f32)], 1),
        jnp.concatenate([jnp.zeros((C1, 1), f32), jnp.ones((C1, 1), f32)], 1),
    ], axis=0)

    xpad = jnp.pad(x[:, 0], ((0, 0), (2, 2), (2, 2)))
    xt = jnp.transpose(xpad, (0, 2, 1)).reshape(P, 2, H + 4, W + 4)

    flops = N * 2 * (H * W * KP * KP * C1 + HP * WP * K2 * C2
                     + HP * WP * K3 * C1 + H * W * KP * KP * C1)
    bytes_accessed = 4 * (xt.size + w1p.size + w2.size + w3p.size + w4p.size
                          + N * (H * W + C2))

    recon_pk, latent = pl.pallas_call(
        _pair_kernel,
        out_shape=(jax.ShapeDtypeStruct((P, 2, H, W), f32),
                   jax.ShapeDtypeStruct((P, 2, C2), f32)),
        grid=(P,),
        in_specs=[
            pl.BlockSpec((None, 2, H + 4, W + 4), lambda p: (p, 0, 0, 0)),
            pl.BlockSpec((KP * KP, CL), lambda p: (0, 0)),
            pl.BlockSpec((1, CL), lambda p: (0, 0)),
            pl.BlockSpec((K2P, C2), lambda p: (0, 0)),
            pl.BlockSpec((1, C2), lambda p: (0, 0)),
            pl.BlockSpec((K3, CL), lambda p: (0, 0)),
            pl.BlockSpec((1, CL), lambda p: (0, 0)),
            pl.BlockSpec((KP * KP, CL), lambda p: (0, 0)),
            pl.BlockSpec((CL, 2), lambda p: (0, 0)),
            pl.BlockSpec(memory_space=pltpu.MemorySpace.SMEM),
        ],
        out_specs=(pl.BlockSpec((None, 2, H, W), lambda p: (p, 0, 0, 0)),
                   pl.BlockSpec((None, 2, C2), lambda p: (p, 0, 0))),
        scratch_shapes=[
            pltpu.VMEM((H + 4, W + 4, CL), f32),
            pltpu.VMEM((WP + 4, WP + 4, CL), f32),
            pltpu.VMEM((HP, WP, CL), jnp.int32),
            pltpu.VMEM((2, HP + 4, WP + 4, C2), f32),
            pltpu.VMEM((2 * HP * WP, K2P), f32),
            pltpu.VMEM((2 * HP * WP, K3), f32),
            pltpu.VMEM((H + 4, W + 4, CL), f32),
            pltpu.VMEM((W, 2 * H), f32),
        ],
        compiler_params=pltpu.CompilerParams(
            dimension_semantics=("parallel",),
            vmem_limit_bytes=32 * 1024 * 1024),
        cost_estimate=pl.CostEstimate(flops=flops, transcendentals=0,
                                      bytes_accessed=bytes_accessed),
    )(xt, w1p, b1p, w2, b2, w3p, b3p, w4p, m2, b4)

    recon = recon_pk.reshape(N, H, W)[:, None, :, :]
    return recon, latent.reshape(-1)


# speedup vs baseline: 2.3960x; 1.0025x over previous
import jax
import jax.numpy as jnp
from jax import lax
from jax.experimental import pallas as pl
from jax.experimental.pallas import tpu as pltpu

KP = 5
PW = 8
H = W = 64
C1, C2 = 64, 128
CL = 2 * C1
HP, WP = H // PW, W // PW
K2 = KP * KP * C1
K2P = 13 * 128
K3 = KP * KP * C2

HIGHEST = jax.lax.Precision.HIGHEST
f32 = jnp.float32


def _pair_kernel(xt_ref, w1_ref, b1_ref, w2_ref, b2_ref, w3_ref, b3_ref,
                 w4_ref, m2_ref, b4_ref, out_ref, lat_ref,
                 xpb, p1pad, idx1, u2pad, imcol2, imcol3, up, scol):
    p1pad[...] = jnp.zeros((WP + 4, WP + 4, CL), f32)
    u2pad[...] = jnp.zeros((2, HP + 4, WP + 4, C2), f32)
    imcol2[:, K2:] = jnp.zeros((2 * HP * WP, K2P - K2), f32)
    up[0:2, :, :] = jnp.zeros((2, W + 4, CL), f32)
    up[H + 2:H + 4, :, :] = jnp.zeros((2, W + 4, CL), f32)
    up[2:2 + H, 0:2, :] = jnp.zeros((H, 2, CL), f32)
    up[2:2 + H, W + 2:W + 4, :] = jnp.zeros((H, 2, CL), f32)

    for v in range(H + 4):
        c0 = jnp.broadcast_to(xt_ref[0][:, v:v + 1], (W + 4, C1))
        c1 = jnp.broadcast_to(xt_ref[1][:, v:v + 1], (W + 4, C1))
        xpb[v, :, :] = jnp.concatenate([c0, c1], axis=-1)

    wcol = lax.broadcasted_iota(jnp.int32, (W, CL), 0)

    for I in range(HP):
        best_v = jnp.full((W, CL), -1.0, f32)
        best_f = jnp.zeros((W, CL), jnp.int32)
        for r in range(PW):
            h = I * PW + r
            acca = jnp.zeros((W, CL), f32)
            accb = jnp.zeros((W, CL), f32)
            for dy in range(KP):
                for dx in range(KP):
                    t = dy * KP + dx
                    prod = xpb[h + dy, dx:dx + W, :] * w1_ref[t, :]
                    if t % 2 == 0:
                        acca = acca + prod
                    else:
                        accb = accb + prod
            a1 = jnp.maximum(acca + accb + b1_ref[0, :], 0.0)
            take = a1 > best_v
            best_f = jnp.where(take, r * W + wcol, best_f)
            best_v = jnp.where(take, a1, best_v)

        v3 = best_v.reshape(WP, PW, CL)
        f3 = best_f.reshape(WP, PW, CL)
        pooled = jnp.max(v3, axis=1)
        sel = jnp.min(jnp.where(v3 == pooled[:, None, :], f3, PW * W), axis=1)
        p1pad[2 + I, 2:2 + WP, :] = pooled
        idx1[I] = sel

    for t in range(KP * KP):
        dy, dx = divmod(t, KP)
        slab = p1pad[dy:dy + HP, dx:dx + WP, :]
        imcol2[0:HP * WP, t * C1:(t + 1) * C1] = (
            slab[:, :, 0:C1].reshape(HP * WP, C1))
        imcol2[HP * WP:2 * HP * WP, t * C1:(t + 1) * C1] = (
            slab[:, :, C1:CL].reshape(HP * WP, C1))
    a2 = jnp.dot(imcol2[...], w2_ref[...],
                 preferred_element_type=f32, precision=HIGHEST) + b2_ref[0, :]
    a2 = jnp.maximum(a2, 0.0)

    piota = lax.broadcasted_iota(jnp.int32, (HP * WP, C2), 0)
    for i in range(2):
        ai = a2[i * HP * WP:(i + 1) * HP * WP, :]
        lat = jnp.max(ai, axis=0)
        lat_ref[i, :] = lat
        idx2 = jnp.min(jnp.where(ai == lat[None, :], piota, HP * WP), axis=0)
        u2 = jnp.where(piota == idx2[None, :], lat[None, :], 0.0)
        u2pad[i, 2:2 + HP, 2:2 + WP, :] = u2.reshape(HP, WP, C2)

    for t in range(KP * KP):
        dy, dx = divmod(t, KP)
        for i in range(2):
            imcol3[i * HP * WP:(i + 1) * HP * WP, t * C2:(t + 1) * C2] = (
                u2pad[i, dy:dy + HP, dx:dx + WP, :].reshape(HP * WP, C2))
    d1 = jnp.dot(imcol3[...], w3_ref[...],
                 preferred_element_type=f32, precision=HIGHEST) + b3_ref[0, :]
    d1 = jnp.maximum(d1, 0.0)
    lane = lax.broadcasted_iota(jnp.int32, (HP * WP, CL), 1)
    d1p = jnp.where(lane < C1, d1[0:HP * WP, :], d1[HP * WP:2 * HP * WP, :])

    riota = lax.broadcasted_iota(jnp.int32, (PW, W, CL), 0)
    wiota = lax.broadcasted_iota(jnp.int32, (PW, W, CL), 1)
    flatpos = riota * W + wiota
    for I in range(HP):
        vals = d1p[I * WP:(I + 1) * WP, :]
        idxs = idx1[I]
        vflat = jnp.broadcast_to(vals[:, None, :], (WP, PW, CL)).reshape(W, CL)
        iflat = jnp.broadcast_to(idxs[:, None, :], (WP, PW, CL)).reshape(W, CL)
        slab = jnp.where(flatpos == iflat[None, :, :], vflat[None, :, :], 0.0)
        up[2 + I * PW:2 + (I + 1) * PW, 2:2 + W, :] = slab

    for h in range(H):
        acca = jnp.zeros((W, CL), f32)
        accb = jnp.zeros((W, CL), f32)
        for dy in range(KP):
            for dx in range(KP):
                t = dy * KP + dx
                prod = up[h + dy, dx:dx + W, :] * w4_ref[t, :]
                if t % 2 == 0:
                    acca = acca + prod
                else:
                    accb = accb + prod
        s = jnp.dot(acca + accb, m2_ref[...],
                    preferred_element_type=f32, precision=HIGHEST)
        scol[:, h:h + 1] = s[:, 0:1]
        scol[:, H + h:H + h + 1] = s[:, 1:2]

    b4 = b4_ref[0]
    sc = scol[...]
    t0 = jnp.maximum(jnp.swapaxes(sc[:, 0:H], 0, 1) + b4, 0.0)
    t1 = jnp.maximum(jnp.swapaxes(sc[:, H:2 * H], 0, 1) + b4, 0.0)
    out_ref[0, :, :] = t0
    out_ref[1, :, :] = t1


def _pack_conv_w(w_oihw):
    co, ci = w_oihw.shape[0], w_oihw.shape[1]
    return jnp.transpose(w_oihw, (2, 3, 1, 0)).reshape(KP * KP * ci, co)


def _pack_convT_w(w_iohw):
    ci, co = w_iohw.shape[0], w_iohw.shape[1]
    return jnp.transpose(w_iohw[:, :, ::-1, ::-1], (2, 3, 0, 1)).reshape(KP * KP * ci, co)


def kernel(x, enc1_w, enc1_b, enc2_w, enc2_b, dec1_w, dec1_b, dec2_w, dec2_b):
    N = x.shape[0]
    P = N // 2

    w1 = _pack_conv_w(enc1_w)
    w1p = jnp.concatenate([w1, w1], axis=1)
    b1p = jnp.tile(enc1_b.reshape(1, C1), (1, 2))
    w2 = jnp.pad(_pack_conv_w(enc2_w), ((0, K2P - K2), (0, 0)))
    b2 = enc2_b.reshape(1, C2)
    w3 = _pack_convT_w(dec1_w)
    w3p = jnp.concatenate([w3, w3], axis=1)
    b3p = jnp.tile(dec1_b.reshape(1, C1), (1, 2))
    w4 = _pack_convT_w(dec2_w).reshape(KP * KP, C1)
    w4p = jnp.concatenate([w4, w4], axis=1)
    b4 = dec2_b.reshape(1)
    m2 = jnp.concatenate([
        jnp.concatenate([jnp.ones((C1, 1), f32), jnp.zeros((C1, 1), f32)], 1),
        jnp.concatenate([jnp.zeros((C1, 1), f32), jnp.ones((C1, 1), f32)], 1),
    ], axis=0)

    xpad = jnp.pad(x[:, 0], ((0, 0), (2, 2), (2, 2)))
    xt = jnp.transpose(xpad, (0, 2, 1)).reshape(P, 2, H + 4, W + 4)

    flops = N * 2 * (H * W * KP * KP * C1 + HP * WP * K2 * C2
                     + HP * WP * K3 * C1 + H * W * KP * KP * C1)
    bytes_accessed = 4 * (xt.size + w1p.size + w2.size + w3p.size + w4p.size
                          + N * (H * W + C2))

    recon_pk, latent = pl.pallas_call(
        _pair_kernel,
        out_shape=(jax.ShapeDtypeStruct((P, 2, H, W), f32),
                   jax.ShapeDtypeStruct((P, 2, C2), f32)),
        grid=(P,),
        in_specs=[
            pl.BlockSpec((None, 2, H + 4, W + 4), lambda p: (p, 0, 0, 0)),
            pl.BlockSpec((KP * KP, CL), lambda p: (0, 0)),
            pl.BlockSpec((1, CL), lambda p: (0, 0)),
            pl.BlockSpec((K2P, C2), lambda p: (0, 0)),
            pl.BlockSpec((1, C2), lambda p: (0, 0)),
            pl.BlockSpec((K3, CL), lambda p: (0, 0)),
            pl.BlockSpec((1, CL), lambda p: (0, 0)),
            pl.BlockSpec((KP * KP, CL), lambda p: (0, 0)),
            pl.BlockSpec((CL, 2), lambda p: (0, 0)),
            pl.BlockSpec(memory_space=pltpu.MemorySpace.SMEM),
        ],
        out_specs=(pl.BlockSpec((None, 2, H, W), lambda p: (p, 0, 0, 0)),
                   pl.BlockSpec((None, 2, C2), lambda p: (p, 0, 0))),
        scratch_shapes=[
            pltpu.VMEM((H + 4, W + 4, CL), f32),
            pltpu.VMEM((WP + 4, WP + 4, CL), f32),
            pltpu.VMEM((HP, WP, CL), jnp.int32),
            pltpu.VMEM((2, HP + 4, WP + 4, C2), f32),
            pltpu.VMEM((2 * HP * WP, K2P), f32),
            pltpu.VMEM((2 * HP * WP, K3), f32),
            pltpu.VMEM((H + 4, W + 4, CL), f32),
            pltpu.VMEM((W, 2 * H), f32),
        ],
        compiler_params=pltpu.CompilerParams(
            dimension_semantics=("parallel",),
            vmem_limit_bytes=32 * 1024 * 1024),
        cost_estimate=pl.CostEstimate(flops=flops, transcendentals=0,
                                      bytes_accessed=bytes_accessed),
    )(xt, w1p, b1p, w2, b2, w3p, b3p, w4p, m2, b4)

    recon = recon_pk.reshape(N, H, W)[:, None, :, :]
    return recon, latent.reshape(-1)
